# Initial kernel scaffold; baseline (speedup 1.0000x reference)
import jax
import jax.numpy as jnp
from jax.experimental import pallas as pl


def kernel(features, edge_index, W0, b0, g0, beta0, W1, b1, g1, beta1, W2, b2, g2, beta2, pW, pb):
    raise NotImplementedError("write your pallas kernel here")


# speedup vs baseline: 1.2502x; 1.2502x over previous
import functools

import jax
import jax.numpy as jnp
from jax import lax
from jax.experimental import pallas as pl
from jax.experimental.pallas import tpu as pltpu
from jax.experimental.pallas import tpu_sc as plsc

NB, NPER, DD = 4000, 9, 128
NN = NB * NPER
EE = 576000
NC, NS = 2, 16
NW = NC * NS
Q = NN // 4
QP = Q + 8
K = 96
EPT = EE // NS
EPW = EE // NW
ZR = 568
R = 288
RC = 1200
RB = 200

_f32 = jnp.float32


def _mesh():
    return plsc.VectorSubcoreMesh(
        core_axis_name="c", subcore_axis_name="s",
        num_cores=NC, num_subcores=NS)


def _per_tile_rows(sid, total, fn):
    @pl.when(sid < NS - 1)
    def _():
        fn(sid * ZR, ZR)

    @pl.when(sid == NS - 1)
    def _():
        fn((NS - 1) * ZR, total - (NS - 1) * ZR)


def _degrees_sc(src, dst):
    @functools.partial(
        pl.kernel,
        out_type=jax.ShapeDtypeStruct((2 * NW, NN), _f32),
        mesh=_mesh(),
        compiler_params=pltpu.CompilerParams(needs_layout_passes=False),
        scratch_types=[
            pltpu.VMEM((K,), jnp.int32),
            pltpu.VMEM((K,), jnp.int32),
            pltpu.VMEM((NN,), _f32),
            pltpu.VMEM((NN,), _f32),
        ],
    )
    def k(src_hbm, dst_hbm, out_hbm, sv, dv, cs, cd):
        cid = lax.axis_index("c")
        sid = lax.axis_index("s")
        wid = sid * NC + cid
        zero16 = jnp.zeros((16,), _f32)
        one16 = jnp.ones((16,), _f32)

        def zb(i, carry):
            cs[pl.ds(i * 16, 16)] = zero16
            cd[pl.ds(i * 16, 16)] = zero16
            return carry

        lax.fori_loop(0, NN // 16, zb, 0)
        ebase = wid * EPW

        def body(b, carry):
            base = ebase + b * K
            pltpu.sync_copy(src_hbm.at[pl.ds(base, K)], sv)
            pltpu.sync_copy(dst_hbm.at[pl.ds(base, K)], dv)
            for j in range(K // 16):
                plsc.addupdate_scatter(cs, [sv[pl.ds(j * 16, 16)]], one16)
                plsc.addupdate_scatter(cd, [dv[pl.ds(j * 16, 16)]], one16)
            return carry

        lax.fori_loop(0, EPW // K, body, 0)
        pltpu.sync_copy(cs, out_hbm.at[wid])
        pltpu.sync_copy(cd, out_hbm.at[NW + wid])

    return k(src, dst)


def _segsum_sc(h, src, dst, zeros):
    @functools.partial(
        pl.kernel,
        out_type=jax.ShapeDtypeStruct((NN, DD), _f32),
        mesh=_mesh(),
        compiler_params=pltpu.CompilerParams(needs_layout_passes=False),
        scratch_types=[
            pltpu.VMEM((K,), jnp.int32),
            pltpu.VMEM((K,), jnp.int32),
            pltpu.VMEM((K, DD), _f32),
            pltpu.VMEM_SHARED((QP, DD), _f32),
            pltpu.SemaphoreType.DMA,
        ],
    )
    def k(h_hbm, src_hbm, dst_hbm, zeros_hbm, out_hbm,
          idx_s, idx_d, rows, acc_sh, sem):
        cid = lax.axis_index("c")
        sid = lax.axis_index("s")
        for r in range(2):
            base = pl.multiple_of((2 * r + cid) * Q, 8)
            _per_tile_rows(sid, QP, lambda off, sz: pltpu.sync_copy(
                zeros_hbm.at[pl.ds(0, sz)], acc_sh.at[pl.ds(off, sz)]))
            plsc.subcore_barrier()
            ebase = sid * EPT

            def body(b, carry):
                bb = ebase + b * K
                pltpu.sync_copy(src_hbm.at[pl.ds(bb, K)], idx_s)
                pltpu.sync_copy(dst_hbm.at[pl.ds(bb, K)], idx_d)
                gat = pltpu.async_copy(h_hbm.at[idx_s], rows, sem)
                for j in range(K // 16):
                    dv = idx_d[pl.ds(j * 16, 16)]
                    ok = (dv >= base) & (dv < base + Q)
                    idx_d[pl.ds(j * 16, 16)] = jnp.where(
                        ok, dv - base, Q + (dv & 7))
                gat.wait()
                pltpu.sync_copy(rows, acc_sh.at[idx_d], add=True)
                return carry

            lax.fori_loop(0, EPT // K, body, 0)
            plsc.subcore_barrier()
            _per_tile_rows(sid, Q, lambda off, sz: pltpu.sync_copy(
                acc_sh.at[pl.ds(off, sz)], out_hbm.at[pl.ds(base + off, sz)]))
            plsc.subcore_barrier()

    return k(h, src, dst, zeros)


def _scales_tc(counts, ones32):
    def body(c0_ref, c1_ref, o_ref, s0_ref, s1_ref):
        dn = (((0,), (0,)), ((), ()))
        c0 = lax.dot_general(c0_ref[...], o_ref[...], dn,
                             preferred_element_type=_f32)
        c1 = lax.dot_general(c1_ref[...], o_ref[...], dn,
                             preferred_element_type=_f32)
        s0_ref[...] = lax.rsqrt(jnp.maximum(c0, 1.0))
        s1_ref[...] = lax.rsqrt(jnp.maximum(c1, 1.0))

    sds = jax.ShapeDtypeStruct((NN, 1), _f32)
    return pl.pallas_call(
        body,
        grid=(1,),
        in_specs=[
            pl.BlockSpec((NW, NN), lambda j: (0, 0)),
            pl.BlockSpec((NW, NN), lambda j: (0, 0)),
            pl.BlockSpec((NW, 1), lambda j: (0, 0)),
        ],
        out_specs=[pl.BlockSpec((NN, 1), lambda j: (0, 0))] * 2,
        out_shape=[sds, sds],
    )(counts.reshape(2, NW, NN)[0], counts.reshape(2, NW, NN)[1], ones32)


def _h0_tc(feats, s_src):
    def body(x_ref, s_ref, o_ref):
        o_ref[...] = x_ref[...] * s_ref[...]

    return pl.pallas_call(
        body,
        grid=(NN // R,),
        in_specs=[
            pl.BlockSpec((R, DD), lambda j: (j, 0)),
            pl.BlockSpec((R, 1), lambda j: (j, 0)),
        ],
        out_specs=pl.BlockSpec((R, DD), lambda j: (j, 0)),
        out_shape=jax.ShapeDtypeStruct((NN, DD), _f32),
    )(feats, s_src)


def _dense_core(a_ref, sd_ref, w_ref, b_ref, g_ref, be_ref):
    x = a_ref[...] * sd_ref[...]
    y = jnp.dot(x, w_ref[...], preferred_element_type=_f32) + b_ref[...]
    mu = jnp.mean(y, axis=1, keepdims=True)
    d = y - mu
    var = jnp.mean(d * d, axis=1, keepdims=True)
    yn = d * lax.rsqrt(var + 1e-5) * g_ref[...] + be_ref[...]
    return jnp.where(yn > 0, yn, jnp.exp(yn) - 1.0)


_row_blk = pl.BlockSpec((R, 1), lambda j: (j, 0))
_mat_blk = pl.BlockSpec((R, DD), lambda j: (j, 0))
_w_blk = pl.BlockSpec((DD, DD), lambda j: (0, 0))
_v_blk = pl.BlockSpec((1, DD), lambda j: (0, 0))


def _layer_tc(agg, s_dst, s_src, W, b, g, be):
    def body(a_ref, sd_ref, ss_ref, w_ref, b_ref, g_ref, be_ref, o_ref):
        e = _dense_core(a_ref, sd_ref, w_ref, b_ref, g_ref, be_ref)
        o_ref[...] = e * ss_ref[...]

    return pl.pallas_call(
        body,
        grid=(NN // R,),
        in_specs=[_mat_blk, _row_blk, _row_blk, _w_blk, _v_blk, _v_blk, _v_blk],
        out_specs=_mat_blk,
        out_shape=jax.ShapeDtypeStruct((NN, DD), _f32),
    )(agg, s_dst, s_src, W, b, g, be)


def _last_layer_tc(agg, s_dst, W, b, g, be, pwt):
    def body(a_ref, sd_ref, w_ref, b_ref, g_ref, be_ref, pwt_ref, o_ref):
        e = _dense_core(a_ref, sd_ref, w_ref, b_ref, g_ref, be_ref)
        o_ref[...] = jnp.sum(e * pwt_ref[...], axis=1, keepdims=True)

    return pl.pallas_call(
        body,
        grid=(NN // R,),
        in_specs=[_mat_blk, _row_blk, _w_blk, _v_blk, _v_blk, _v_blk,
                  pl.BlockSpec((R, DD), lambda j: (0, 0))],
        out_specs=_row_blk,
        out_shape=jax.ShapeDtypeStruct((NN, 1), _f32),
    )(agg, s_dst, W, b, g, be, pwt)


def _head_tc(rowsums, pb):
    def body(x_ref, pb_ref, o_ref):
        o_ref[...] = jnp.sum(x_ref[...], axis=1, keepdims=True) + pb_ref[...]

    return pl.pallas_call(
        body,
        grid=(NB // RB,),
        in_specs=[pl.BlockSpec((RB, NPER), lambda j: (j, 0)),
                  pl.BlockSpec((1, 1), lambda j: (0, 0))],
        out_specs=pl.BlockSpec((RB, 1), lambda j: (j, 0)),
        out_shape=jax.ShapeDtypeStruct((NB, 1), _f32),
    )(rowsums, pb)


def kernel(features, edge_index, W0, b0, g0, beta0, W1, b1, g1, beta1,
           W2, b2, g2, beta2, pW, pb):
    src, dst = edge_index[0], edge_index[1]
    feats = features.reshape(NN, DD)
    zeros = jnp.zeros((ZR, DD), _f32)
    ones32 = jnp.ones((NW, 1), _f32)
    pwt = jnp.tile(pW.reshape(NPER, DD), (R // NPER, 1))

    counts = _degrees_sc(src, dst)
    s_src, s_dst = _scales_tc(counts, ones32)

    h = _h0_tc(feats, s_src)
    for W, b, g, be in ((W0, b0, g0, beta0), (W1, b1, g1, beta1)):
        agg = _segsum_sc(h, src, dst, zeros)
        h = _layer_tc(agg, s_dst, s_src, W, b.reshape(1, DD),
                      g.reshape(1, DD), be.reshape(1, DD))
    agg = _segsum_sc(h, src, dst, zeros)
    rowsums = _last_layer_tc(agg, s_dst, W2, b2.reshape(1, DD),
                             g2.reshape(1, DD), beta2.reshape(1, DD), pwt)
    return _head_tc(rowsums.reshape(NB, NPER), pb.reshape(1, 1))


# speedup vs baseline: 1.6978x; 1.3580x over previous
import functools

import jax
import jax.numpy as jnp
from jax import lax
from jax.experimental import pallas as pl
from jax.experimental.pallas import tpu as pltpu
from jax.experimental.pallas import tpu_sc as plsc

NB, NPER, DD = 4000, 9, 128
NN = NB * NPER
EE = 576000
NC, NS = 2, 16
NW = NC * NS
Q = NN // 4
QP = Q + 8
K = 96
EPT = EE // NS
EPW = EE // NW
ZR = 568
R = 288
RC = 1200
RB = 200

_f32 = jnp.float32


def _mesh():
    return plsc.VectorSubcoreMesh(
        core_axis_name="c", subcore_axis_name="s",
        num_cores=NC, num_subcores=NS)


def _per_tile_rows(sid, total, fn):
    @pl.when(sid < NS - 1)
    def _():
        fn(sid * ZR, ZR)

    @pl.when(sid == NS - 1)
    def _():
        fn((NS - 1) * ZR, total - (NS - 1) * ZR)


def _degrees_sc(src, dst):
    @functools.partial(
        pl.kernel,
        out_type=jax.ShapeDtypeStruct((2 * NW, NN), _f32),
        mesh=_mesh(),
        compiler_params=pltpu.CompilerParams(needs_layout_passes=False),
        scratch_types=[
            pltpu.VMEM((K,), jnp.int32),
            pltpu.VMEM((K,), jnp.int32),
            pltpu.VMEM((NN,), _f32),
            pltpu.VMEM((NN,), _f32),
        ],
    )
    def k(src_hbm, dst_hbm, out_hbm, sv, dv, cs, cd):
        cid = lax.axis_index("c")
        sid = lax.axis_index("s")
        wid = sid * NC + cid
        zero16 = jnp.zeros((16,), _f32)
        one16 = jnp.ones((16,), _f32)

        def zb(i, carry):
            cs[pl.ds(i * 16, 16)] = zero16
            cd[pl.ds(i * 16, 16)] = zero16
            return carry

        lax.fori_loop(0, NN // 16, zb, 0)
        ebase = wid * EPW

        def body(b, carry):
            base = ebase + b * K
            pltpu.sync_copy(src_hbm.at[pl.ds(base, K)], sv)
            pltpu.sync_copy(dst_hbm.at[pl.ds(base, K)], dv)
            for j in range(K // 16):
                plsc.addupdate_scatter(cs, [sv[pl.ds(j * 16, 16)]], one16)
                plsc.addupdate_scatter(cd, [dv[pl.ds(j * 16, 16)]], one16)
            return carry

        lax.fori_loop(0, EPW // K, body, 0)
        pltpu.sync_copy(cs, out_hbm.at[wid])
        pltpu.sync_copy(cd, out_hbm.at[NW + wid])

    return k(src, dst)


def _segsum_sc(h, src, dst, zeros):
    @functools.partial(
        pl.kernel,
        out_type=jax.ShapeDtypeStruct((NN, DD), _f32),
        mesh=_mesh(),
        compiler_params=pltpu.CompilerParams(needs_layout_passes=False),
        scratch_types=[
            pltpu.VMEM((K,), jnp.int32),
            pltpu.VMEM((K,), jnp.int32),
            pltpu.VMEM((2 * K,), jnp.int32),
            pltpu.VMEM((2 * K,), jnp.int32),
            pltpu.VMEM((K,), jnp.int32),
            pltpu.VMEM((K,), jnp.int32),
            pltpu.VMEM((K, DD), _f32),
            pltpu.VMEM_SHARED((QP, DD), _f32),
            pltpu.SemaphoreType.DMA,
        ],
    )
    def k(h_hbm, src_hbm, dst_hbm, zeros_hbm, out_hbm,
          sv, dv, ssta, dsta, sidx_f, didx_f, rows, acc_sh, sem):
        cid = lax.axis_index("c")
        sid = lax.axis_index("s")
        G = K // 16

        def fire(p):
            for j in range(G):
                sidx_f[pl.ds(j * 16, 16)] = ssta[pl.ds(j * 16, 16)]
                didx_f[pl.ds(j * 16, 16)] = dsta[pl.ds(j * 16, 16)]
            pltpu.async_copy(h_hbm.at[sidx_f], rows, sem).wait()
            pltpu.sync_copy(rows, acc_sh.at[didx_f], add=True)
            for j in range(G):
                ssta[pl.ds(j * 16, 16)] = ssta[pl.ds(K + j * 16, 16)]
                dsta[pl.ds(j * 16, 16)] = dsta[pl.ds(K + j * 16, 16)]
            return p - K

        for r in range(2):
            base = pl.multiple_of((2 * r + cid) * Q, 8)
            _per_tile_rows(sid, QP, lambda off, sz: pltpu.sync_copy(
                zeros_hbm.at[pl.ds(0, sz)], acc_sh.at[pl.ds(off, sz)]))
            plsc.subcore_barrier()
            ebase = sid * EPT

            def body(b, pos):
                bb = ebase + b * K
                pltpu.sync_copy(src_hbm.at[pl.ds(bb, K)], sv)
                pltpu.sync_copy(dst_hbm.at[pl.ds(bb, K)], dv)
                for j in range(G):
                    s16 = sv[pl.ds(j * 16, 16)]
                    d16 = dv[pl.ds(j * 16, 16)]
                    ok = (d16 >= base) & (d16 < base + Q)
                    plsc.store_compressed(ssta.at[pl.ds(pos, 16)], s16,
                                          mask=ok)
                    plsc.store_compressed(dsta.at[pl.ds(pos, 16)], d16 - base,
                                          mask=ok)
                    pos = pos + jnp.sum(ok.astype(jnp.int32))
                return lax.cond(pos >= K, fire, lambda p: p, pos)

            pos = lax.fori_loop(0, EPT // K, body, 0)
            for j in range(2 * G):
                lane = lax.iota(jnp.int32, 16) + j * 16
                pad = lane >= pos
                ssta[pl.ds(j * 16, 16)] = jnp.where(
                    pad, 0, ssta[pl.ds(j * 16, 16)])
                dsta[pl.ds(j * 16, 16)] = jnp.where(
                    pad, Q, dsta[pl.ds(j * 16, 16)])
            fire(0)
            fire(0)
            plsc.subcore_barrier()
            _per_tile_rows(sid, Q, lambda off, sz: pltpu.sync_copy(
                acc_sh.at[pl.ds(off, sz)], out_hbm.at[pl.ds(base + off, sz)]))
            plsc.subcore_barrier()

    return k(h, src, dst, zeros)


def _scales_tc(counts, ones32):
    def body(c0_ref, c1_ref, o_ref, s0_ref, s1_ref):
        dn = (((0,), (0,)), ((), ()))
        c0 = lax.dot_general(c0_ref[...], o_ref[...], dn,
                             preferred_element_type=_f32)
        c1 = lax.dot_general(c1_ref[...], o_ref[...], dn,
                             preferred_element_type=_f32)
        s0_ref[...] = lax.rsqrt(jnp.maximum(c0, 1.0))
        s1_ref[...] = lax.rsqrt(jnp.maximum(c1, 1.0))

    sds = jax.ShapeDtypeStruct((NN, 1), _f32)
    return pl.pallas_call(
        body,
        grid=(1,),
        in_specs=[
            pl.BlockSpec((NW, NN), lambda j: (0, 0)),
            pl.BlockSpec((NW, NN), lambda j: (0, 0)),
            pl.BlockSpec((NW, 1), lambda j: (0, 0)),
        ],
        out_specs=[pl.BlockSpec((NN, 1), lambda j: (0, 0))] * 2,
        out_shape=[sds, sds],
    )(counts.reshape(2, NW, NN)[0], counts.reshape(2, NW, NN)[1], ones32)


def _h0_tc(feats, s_src):
    def body(x_ref, s_ref, o_ref):
        o_ref[...] = x_ref[...] * s_ref[...]

    return pl.pallas_call(
        body,
        grid=(NN // R,),
        in_specs=[
            pl.BlockSpec((R, DD), lambda j: (j, 0)),
            pl.BlockSpec((R, 1), lambda j: (j, 0)),
        ],
        out_specs=pl.BlockSpec((R, DD), lambda j: (j, 0)),
        out_shape=jax.ShapeDtypeStruct((NN, DD), _f32),
    )(feats, s_src)


def _dense_core(a_ref, sd_ref, w_ref, b_ref, g_ref, be_ref):
    x = a_ref[...] * sd_ref[...]
    y = jnp.dot(x, w_ref[...], preferred_element_type=_f32) + b_ref[...]
    mu = jnp.mean(y, axis=1, keepdims=True)
    d = y - mu
    var = jnp.mean(d * d, axis=1, keepdims=True)
    yn = d * lax.rsqrt(var + 1e-5) * g_ref[...] + be_ref[...]
    return jnp.where(yn > 0, yn, jnp.exp(yn) - 1.0)


_row_blk = pl.BlockSpec((R, 1), lambda j: (j, 0))
_mat_blk = pl.BlockSpec((R, DD), lambda j: (j, 0))
_w_blk = pl.BlockSpec((DD, DD), lambda j: (0, 0))
_v_blk = pl.BlockSpec((1, DD), lambda j: (0, 0))


def _layer_tc(agg, s_dst, s_src, W, b, g, be):
    def body(a_ref, sd_ref, ss_ref, w_ref, b_ref, g_ref, be_ref, o_ref):
        e = _dense_core(a_ref, sd_ref, w_ref, b_ref, g_ref, be_ref)
        o_ref[...] = e * ss_ref[...]

    return pl.pallas_call(
        body,
        grid=(NN // R,),
        in_specs=[_mat_blk, _row_blk, _row_blk, _w_blk, _v_blk, _v_blk, _v_blk],
        out_specs=_mat_blk,
        out_shape=jax.ShapeDtypeStruct((NN, DD), _f32),
    )(agg, s_dst, s_src, W, b, g, be)


def _last_layer_tc(agg, s_dst, W, b, g, be, pwt):
    def body(a_ref, sd_ref, w_ref, b_ref, g_ref, be_ref, pwt_ref, o_ref):
        e = _dense_core(a_ref, sd_ref, w_ref, b_ref, g_ref, be_ref)
        o_ref[...] = jnp.sum(e * pwt_ref[...], axis=1, keepdims=True)

    return pl.pallas_call(
        body,
        grid=(NN // R,),
        in_specs=[_mat_blk, _row_blk, _w_blk, _v_blk, _v_blk, _v_blk,
                  pl.BlockSpec((R, DD), lambda j: (0, 0))],
        out_specs=_row_blk,
        out_shape=jax.ShapeDtypeStruct((NN, 1), _f32),
    )(agg, s_dst, W, b, g, be, pwt)


def _head_tc(rowsums, pb):
    def body(x_ref, pb_ref, o_ref):
        o_ref[...] = jnp.sum(x_ref[...], axis=1, keepdims=True) + pb_ref[...]

    return pl.pallas_call(
        body,
        grid=(NB // RB,),
        in_specs=[pl.BlockSpec((RB, NPER), lambda j: (j, 0)),
                  pl.BlockSpec((1, 1), lambda j: (0, 0))],
        out_specs=pl.BlockSpec((RB, 1), lambda j: (j, 0)),
        out_shape=jax.ShapeDtypeStruct((NB, 1), _f32),
    )(rowsums, pb)


def kernel(features, edge_index, W0, b0, g0, beta0, W1, b1, g1, beta1,
           W2, b2, g2, beta2, pW, pb):
    src, dst = edge_index[0], edge_index[1]
    feats = features.reshape(NN, DD)
    zeros = jnp.zeros((ZR, DD), _f32)
    ones32 = jnp.ones((NW, 1), _f32)
    pwt = jnp.tile(pW.reshape(NPER, DD), (R // NPER, 1))

    counts = _degrees_sc(src, dst)
    s_src, s_dst = _scales_tc(counts, ones32)

    h = _h0_tc(feats, s_src)
    for W, b, g, be in ((W0, b0, g0, beta0), (W1, b1, g1, beta1)):
        agg = _segsum_sc(h, src, dst, zeros)
        h = _layer_tc(agg, s_dst, s_src, W, b.reshape(1, DD),
                      g.reshape(1, DD), be.reshape(1, DD))
    agg = _segsum_sc(h, src, dst, zeros)
    rowsums = _last_layer_tc(agg, s_dst, W2, b2.reshape(1, DD),
                             g2.reshape(1, DD), beta2.reshape(1, DD), pwt)
    return _head_tc(rowsums.reshape(NB, NPER), pb.reshape(1, 1))


# speedup vs baseline: 2.6486x; 1.5600x over previous
import functools

import jax
import jax.numpy as jnp
from jax import lax
from jax.experimental import pallas as pl
from jax.experimental.pallas import tpu as pltpu
from jax.experimental.pallas import tpu_sc as plsc

NB, NPER, DD = 4000, 9, 128
NN = NB * NPER
EE = 576000
NC, NS = 2, 16
NW = NC * NS
Q = NN // 4
QP = Q + 8
K = 96
FG = 3
CAP = K + 16 * FG
IB = 1200
EPT = EE // NS
EPW = EE // NW
ZR = 568
R = 288
RC = 1200
RB = 200

_f32 = jnp.float32


def _mesh():
    return plsc.VectorSubcoreMesh(
        core_axis_name="c", subcore_axis_name="s",
        num_cores=NC, num_subcores=NS)


def _per_tile_rows(sid, total, fn):
    @pl.when(sid < NS - 1)
    def _():
        fn(sid * ZR, ZR)

    @pl.when(sid == NS - 1)
    def _():
        fn((NS - 1) * ZR, total - (NS - 1) * ZR)


def _degrees_sc(src, dst):
    @functools.partial(
        pl.kernel,
        out_type=jax.ShapeDtypeStruct((2 * NW, NN), _f32),
        mesh=_mesh(),
        compiler_params=pltpu.CompilerParams(needs_layout_passes=False),
        scratch_types=[
            pltpu.VMEM((K,), jnp.int32),
            pltpu.VMEM((K,), jnp.int32),
            pltpu.VMEM((NN,), _f32),
            pltpu.VMEM((NN,), _f32),
        ],
    )
    def k(src_hbm, dst_hbm, out_hbm, sv, dv, cs, cd):
        cid = lax.axis_index("c")
        sid = lax.axis_index("s")
        wid = sid * NC + cid
        zero16 = jnp.zeros((16,), _f32)
        one16 = jnp.ones((16,), _f32)

        def zb(i, carry):
            cs[pl.ds(i * 16, 16)] = zero16
            cd[pl.ds(i * 16, 16)] = zero16
            return carry

        lax.fori_loop(0, NN // 16, zb, 0)
        ebase = wid * EPW

        def body(b, carry):
            base = ebase + b * K
            pltpu.sync_copy(src_hbm.at[pl.ds(base, K)], sv)
            pltpu.sync_copy(dst_hbm.at[pl.ds(base, K)], dv)
            for j in range(K // 16):
                plsc.addupdate_scatter(cs, [sv[pl.ds(j * 16, 16)]], one16)
                plsc.addupdate_scatter(cd, [dv[pl.ds(j * 16, 16)]], one16)
            return carry

        lax.fori_loop(0, EPW // K, body, 0)
        pltpu.sync_copy(cs, out_hbm.at[wid])
        pltpu.sync_copy(cd, out_hbm.at[NW + wid])

    return k(src, dst)


def _segsum_sc(h, src, dst, zeros):
    @functools.partial(
        pl.kernel,
        out_type=jax.ShapeDtypeStruct((NN, DD), _f32),
        mesh=_mesh(),
        compiler_params=pltpu.CompilerParams(needs_layout_passes=False),
        scratch_types=[
            pltpu.VMEM((IB,), jnp.int32),
            pltpu.VMEM((IB,), jnp.int32),
            pltpu.VMEM((IB,), jnp.int32),
            pltpu.VMEM((IB,), jnp.int32),
            pltpu.VMEM((CAP,), jnp.int32),
            pltpu.VMEM((CAP,), jnp.int32),
            pltpu.VMEM((K,), jnp.int32),
            pltpu.VMEM((K,), jnp.int32),
            pltpu.VMEM((K, DD), _f32),
            pltpu.VMEM_SHARED((QP, DD), _f32),
            pltpu.SemaphoreType.DMA,
            pltpu.SemaphoreType.DMA,
        ],
    )
    def k(h_hbm, src_hbm, dst_hbm, zeros_hbm, out_hbm,
          sv0, dv0, sv1, dv1, ssta, dsta, sidx_f, didx_f, rows, acc_sh,
          sem, semi):
        cid = lax.axis_index("c")
        sid = lax.axis_index("s")
        G = K // 16

        def fire(p):
            for j in range(G):
                sidx_f[pl.ds(j * 16, 16)] = ssta[pl.ds(j * 16, 16)]
                didx_f[pl.ds(j * 16, 16)] = dsta[pl.ds(j * 16, 16)]
            pltpu.async_copy(h_hbm.at[sidx_f], rows, sem).wait()
            pltpu.sync_copy(rows, acc_sh.at[didx_f], add=True)
            for j in range((CAP - K) // 16):
                ssta[pl.ds(j * 16, 16)] = ssta[pl.ds(K + j * 16, 16)]
                dsta[pl.ds(j * 16, 16)] = dsta[pl.ds(K + j * 16, 16)]
            return p - K

        def pad_tail(limit):
            for j in range(CAP // 16):
                lane = lax.iota(jnp.int32, 16) + j * 16
                pad = lane >= limit
                ssta[pl.ds(j * 16, 16)] = jnp.where(
                    pad, 0, ssta[pl.ds(j * 16, 16)])
                dsta[pl.ds(j * 16, 16)] = jnp.where(
                    pad, Q, dsta[pl.ds(j * 16, 16)])

        def issue(b, s_buf, d_buf, ebase):
            bb = ebase + b * IB
            pltpu.async_copy(src_hbm.at[pl.ds(bb, IB)], s_buf, semi)
            pltpu.async_copy(dst_hbm.at[pl.ds(bb, IB)], d_buf, semi)

        def wait2():
            pltpu.make_async_copy(src_hbm.at[pl.ds(0, IB)], sv0, semi).wait()
            pltpu.make_async_copy(src_hbm.at[pl.ds(0, IB)], dv0, semi).wait()

        def process(s_buf, d_buf, pos, base):
            for jj in range(IB // (16 * FG)):
                for j in range(FG):
                    o = (jj * FG + j) * 16
                    s16 = s_buf[pl.ds(o, 16)]
                    d16 = d_buf[pl.ds(o, 16)]
                    ok = (d16 >= base) & (d16 < base + Q)
                    plsc.store_compressed(ssta.at[pl.ds(pos, 16)], s16,
                                          mask=ok)
                    plsc.store_compressed(dsta.at[pl.ds(pos, 16)],
                                          d16 - base, mask=ok)
                    pos = pos + jnp.sum(ok.astype(jnp.int32))
                pos = lax.cond(pos >= K, fire, lambda p: p, pos)
            return pos

        NSB = EPT // IB
        for r in range(2):
            base = pl.multiple_of((2 * r + cid) * Q, 8)
            _per_tile_rows(sid, QP, lambda off, sz: pltpu.sync_copy(
                zeros_hbm.at[pl.ds(0, sz)], acc_sh.at[pl.ds(off, sz)]))
            ebase = sid * EPT
            issue(0, sv0, dv0, ebase)
            plsc.subcore_barrier()

            def pair(g, pos):
                b0 = 2 * g
                wait2()
                issue(b0 + 1, sv1, dv1, ebase)
                pos = process(sv0, dv0, pos, base)
                wait2()

                @pl.when(b0 + 2 < NSB)
                def _():
                    issue(b0 + 2, sv0, dv0, ebase)

                return process(sv1, dv1, pos, base)

            pos = lax.fori_loop(0, NSB // 2, pair, 0)
            pad_tail(pos)
            pos = fire(pos)
            pad_tail(jnp.maximum(pos, 0))
            fire(0)
            plsc.subcore_barrier()
            _per_tile_rows(sid, Q, lambda off, sz: pltpu.sync_copy(
                acc_sh.at[pl.ds(off, sz)], out_hbm.at[pl.ds(base + off, sz)]))
            plsc.subcore_barrier()

    return k(h, src, dst, zeros)


def _scales_tc(counts, ones32):
    def body(c0_ref, c1_ref, o_ref, s0_ref, s1_ref):
        dn = (((0,), (0,)), ((), ()))
        c0 = lax.dot_general(c0_ref[...], o_ref[...], dn,
                             preferred_element_type=_f32)
        c1 = lax.dot_general(c1_ref[...], o_ref[...], dn,
                             preferred_element_type=_f32)
        s0_ref[...] = lax.rsqrt(jnp.maximum(c0, 1.0))
        s1_ref[...] = lax.rsqrt(jnp.maximum(c1, 1.0))

    sds = jax.ShapeDtypeStruct((NN, 1), _f32)
    return pl.pallas_call(
        body,
        grid=(1,),
        in_specs=[
            pl.BlockSpec((NW, NN), lambda j: (0, 0)),
            pl.BlockSpec((NW, NN), lambda j: (0, 0)),
            pl.BlockSpec((NW, 1), lambda j: (0, 0)),
        ],
        out_specs=[pl.BlockSpec((NN, 1), lambda j: (0, 0))] * 2,
        out_shape=[sds, sds],
    )(counts.reshape(2, NW, NN)[0], counts.reshape(2, NW, NN)[1], ones32)


def _h0_tc(feats, s_src):
    def body(x_ref, s_ref, o_ref):
        o_ref[...] = x_ref[...] * s_ref[...]

    return pl.pallas_call(
        body,
        grid=(NN // R,),
        in_specs=[
            pl.BlockSpec((R, DD), lambda j: (j, 0)),
            pl.BlockSpec((R, 1), lambda j: (j, 0)),
        ],
        out_specs=pl.BlockSpec((R, DD), lambda j: (j, 0)),
        out_shape=jax.ShapeDtypeStruct((NN, DD), _f32),
    )(feats, s_src)


def _dense_core(a_ref, sd_ref, w_ref, b_ref, g_ref, be_ref):
    x = a_ref[...] * sd_ref[...]
    y = jnp.dot(x, w_ref[...], preferred_element_type=_f32) + b_ref[...]
    mu = jnp.mean(y, axis=1, keepdims=True)
    d = y - mu
    var = jnp.mean(d * d, axis=1, keepdims=True)
    yn = d * lax.rsqrt(var + 1e-5) * g_ref[...] + be_ref[...]
    return jnp.where(yn > 0, yn, jnp.exp(yn) - 1.0)


_row_blk = pl.BlockSpec((R, 1), lambda j: (j, 0))
_mat_blk = pl.BlockSpec((R, DD), lambda j: (j, 0))
_w_blk = pl.BlockSpec((DD, DD), lambda j: (0, 0))
_v_blk = pl.BlockSpec((1, DD), lambda j: (0, 0))


def _layer_tc(agg, s_dst, s_src, W, b, g, be):
    def body(a_ref, sd_ref, ss_ref, w_ref, b_ref, g_ref, be_ref, o_ref):
        e = _dense_core(a_ref, sd_ref, w_ref, b_ref, g_ref, be_ref)
        o_ref[...] = e * ss_ref[...]

    return pl.pallas_call(
        body,
        grid=(NN // R,),
        in_specs=[_mat_blk, _row_blk, _row_blk, _w_blk, _v_blk, _v_blk, _v_blk],
        out_specs=_mat_blk,
        out_shape=jax.ShapeDtypeStruct((NN, DD), _f32),
    )(agg, s_dst, s_src, W, b, g, be)


def _last_layer_tc(agg, s_dst, W, b, g, be, pwt):
    def body(a_ref, sd_ref, w_ref, b_ref, g_ref, be_ref, pwt_ref, o_ref):
        e = _dense_core(a_ref, sd_ref, w_ref, b_ref, g_ref, be_ref)
        o_ref[...] = jnp.sum(e * pwt_ref[...], axis=1, keepdims=True)

    return pl.pallas_call(
        body,
        grid=(NN // R,),
        in_specs=[_mat_blk, _row_blk, _w_blk, _v_blk, _v_blk, _v_blk,
                  pl.BlockSpec((R, DD), lambda j: (0, 0))],
        out_specs=_row_blk,
        out_shape=jax.ShapeDtypeStruct((NN, 1), _f32),
    )(agg, s_dst, W, b, g, be, pwt)


def _head_tc(rowsums, pb):
    def body(x_ref, pb_ref, o_ref):
        o_ref[...] = jnp.sum(x_ref[...], axis=1, keepdims=True) + pb_ref[...]

    return pl.pallas_call(
        body,
        grid=(NB // RB,),
        in_specs=[pl.BlockSpec((RB, NPER), lambda j: (j, 0)),
                  pl.BlockSpec((1, 1), lambda j: (0, 0))],
        out_specs=pl.BlockSpec((RB, 1), lambda j: (j, 0)),
        out_shape=jax.ShapeDtypeStruct((NB, 1), _f32),
    )(rowsums, pb)


def kernel(features, edge_index, W0, b0, g0, beta0, W1, b1, g1, beta1,
           W2, b2, g2, beta2, pW, pb):
    src, dst = edge_index[0], edge_index[1]
    feats = features.reshape(NN, DD)
    zeros = jnp.zeros((ZR, DD), _f32)
    ones32 = jnp.ones((NW, 1), _f32)
    pwt = jnp.tile(pW.reshape(NPER, DD), (R // NPER, 1))

    counts = _degrees_sc(src, dst)
    s_src, s_dst = _scales_tc(counts, ones32)

    h = _h0_tc(feats, s_src)
    for W, b, g, be in ((W0, b0, g0, beta0), (W1, b1, g1, beta1)):
        agg = _segsum_sc(h, src, dst, zeros)
        h = _layer_tc(agg, s_dst, s_src, W, b.reshape(1, DD),
                      g.reshape(1, DD), be.reshape(1, DD))
    agg = _segsum_sc(h, src, dst, zeros)
    rowsums = _last_layer_tc(agg, s_dst, W2, b2.reshape(1, DD),
                             g2.reshape(1, DD), beta2.reshape(1, DD), pwt)
    return _head_tc(rowsums.reshape(NB, NPER), pb.reshape(1, 1))


# speedup vs baseline: 2.8636x; 1.0812x over previous
import functools

import jax
import jax.numpy as jnp
from jax import lax
from jax.experimental import pallas as pl
from jax.experimental.pallas import tpu as pltpu
from jax.experimental.pallas import tpu_sc as plsc

NB, NPER, DD = 4000, 9, 128
NN = NB * NPER
EE = 576000
NC, NS = 2, 16
NW = NC * NS
Q = NN // 4
QP = Q + 8
K = 96
FG = 5
CAP = K + 16 * FG
IB = 1200
EPT = EE // NS
EPW = EE // NW
ZR = 568
R = 288
RC = 1200
RB = 200

_f32 = jnp.float32


def _mesh():
    return plsc.VectorSubcoreMesh(
        core_axis_name="c", subcore_axis_name="s",
        num_cores=NC, num_subcores=NS)


def _per_tile_rows(sid, total, fn):
    @pl.when(sid < NS - 1)
    def _():
        fn(sid * ZR, ZR)

    @pl.when(sid == NS - 1)
    def _():
        fn((NS - 1) * ZR, total - (NS - 1) * ZR)


def _degrees_sc(src, dst):
    @functools.partial(
        pl.kernel,
        out_type=jax.ShapeDtypeStruct((2 * NW, NN), _f32),
        mesh=_mesh(),
        compiler_params=pltpu.CompilerParams(needs_layout_passes=False),
        scratch_types=[
            pltpu.VMEM((K,), jnp.int32),
            pltpu.VMEM((K,), jnp.int32),
            pltpu.VMEM((NN,), _f32),
            pltpu.VMEM((NN,), _f32),
        ],
    )
    def k(src_hbm, dst_hbm, out_hbm, sv, dv, cs, cd):
        cid = lax.axis_index("c")
        sid = lax.axis_index("s")
        wid = sid * NC + cid
        zero16 = jnp.zeros((16,), _f32)
        one16 = jnp.ones((16,), _f32)

        def zb(i, carry):
            cs[pl.ds(i * 16, 16)] = zero16
            cd[pl.ds(i * 16, 16)] = zero16
            return carry

        lax.fori_loop(0, NN // 16, zb, 0)
        ebase = wid * EPW

        def body(b, carry):
            base = ebase + b * K
            pltpu.sync_copy(src_hbm.at[pl.ds(base, K)], sv)
            pltpu.sync_copy(dst_hbm.at[pl.ds(base, K)], dv)
            for j in range(K // 16):
                plsc.addupdate_scatter(cs, [sv[pl.ds(j * 16, 16)]], one16)
                plsc.addupdate_scatter(cd, [dv[pl.ds(j * 16, 16)]], one16)
            return carry

        lax.fori_loop(0, EPW // K, body, 0)
        pltpu.sync_copy(cs, out_hbm.at[wid])
        pltpu.sync_copy(cd, out_hbm.at[NW + wid])

    return k(src, dst)


def _segsum_sc(h, src, dst, zeros):
    @functools.partial(
        pl.kernel,
        out_type=jax.ShapeDtypeStruct((NN, DD), _f32),
        mesh=_mesh(),
        compiler_params=pltpu.CompilerParams(needs_layout_passes=False),
        scratch_types=[
            pltpu.VMEM((IB,), jnp.int32),
            pltpu.VMEM((IB,), jnp.int32),
            pltpu.VMEM((IB,), jnp.int32),
            pltpu.VMEM((IB,), jnp.int32),
            pltpu.VMEM((CAP,), jnp.int32),
            pltpu.VMEM((CAP,), jnp.int32),
            pltpu.VMEM((K,), jnp.int32),
            pltpu.VMEM((K,), jnp.int32),
            pltpu.VMEM((K,), jnp.int32),
            pltpu.VMEM((K,), jnp.int32),
            pltpu.VMEM((K, DD), _f32),
            pltpu.VMEM((K, DD), _f32),
            pltpu.VMEM_SHARED((QP, DD), _f32),
            pltpu.SemaphoreType.DMA,
            pltpu.SemaphoreType.DMA,
            pltpu.SemaphoreType.DMA,
            pltpu.SemaphoreType.DMA,
        ],
    )
    def k(h_hbm, src_hbm, dst_hbm, zeros_hbm, out_hbm,
          sv0, dv0, sv1, dv1, ssta, dsta, sidx_f0, didx_f0, sidx_f1, didx_f1,
          rows0, rows1, acc_sh, semg, semi, sems0, sems1):
        cid = lax.axis_index("c")
        sid = lax.axis_index("s")
        G = K // 16

        def fire(args):
            p, fc = args

            def slot(sidx_f, didx_f, rows, sems):
                @pl.when(fc >= 2)
                def _():
                    pltpu.make_async_copy(
                        rows, acc_sh.at[didx_f], sems).wait()

                for j in range(G):
                    sidx_f[pl.ds(j * 16, 16)] = ssta[pl.ds(j * 16, 16)]
                    didx_f[pl.ds(j * 16, 16)] = dsta[pl.ds(j * 16, 16)]
                pltpu.async_copy(h_hbm.at[sidx_f], rows, semg).wait()
                pltpu.async_copy(rows, acc_sh.at[didx_f], sems, add=True)

            @pl.when((fc & 1) == 0)
            def _():
                slot(sidx_f0, didx_f0, rows0, sems0)

            @pl.when((fc & 1) == 1)
            def _():
                slot(sidx_f1, didx_f1, rows1, sems1)

            for j in range((CAP - K) // 16):
                ssta[pl.ds(j * 16, 16)] = ssta[pl.ds(K + j * 16, 16)]
                dsta[pl.ds(j * 16, 16)] = dsta[pl.ds(K + j * 16, 16)]
            return p - K, fc + 1

        def pad_tail(limit):
            for j in range(CAP // 16):
                lane = lax.iota(jnp.int32, 16) + j * 16
                pad = lane >= limit
                ssta[pl.ds(j * 16, 16)] = jnp.where(
                    pad, 0, ssta[pl.ds(j * 16, 16)])
                dsta[pl.ds(j * 16, 16)] = jnp.where(
                    pad, Q, dsta[pl.ds(j * 16, 16)])

        def issue(b, s_buf, d_buf, ebase):
            bb = ebase + b * IB
            pltpu.async_copy(src_hbm.at[pl.ds(bb, IB)], s_buf, semi)
            pltpu.async_copy(dst_hbm.at[pl.ds(bb, IB)], d_buf, semi)

        def wait2():
            pltpu.make_async_copy(src_hbm.at[pl.ds(0, IB)], sv0, semi).wait()
            pltpu.make_async_copy(src_hbm.at[pl.ds(0, IB)], dv0, semi).wait()

        def process(s_buf, d_buf, state, base):
            for jj in range(IB // (16 * FG)):
                pos, fc = state
                for j in range(FG):
                    o = (jj * FG + j) * 16
                    s16 = s_buf[pl.ds(o, 16)]
                    d16 = d_buf[pl.ds(o, 16)]
                    ok = (d16 >= base) & (d16 < base + Q)
                    plsc.store_compressed(ssta.at[pl.ds(pos, 16)], s16,
                                          mask=ok)
                    plsc.store_compressed(dsta.at[pl.ds(pos, 16)],
                                          d16 - base, mask=ok)
                    pos = pos + jnp.sum(ok.astype(jnp.int32))
                state = lax.cond(pos >= K, fire, lambda a: a, (pos, fc))
            return state

        NSB = EPT // IB
        for r in range(2):
            base = pl.multiple_of((2 * r + cid) * Q, 8)
            _per_tile_rows(sid, QP, lambda off, sz: pltpu.sync_copy(
                zeros_hbm.at[pl.ds(0, sz)], acc_sh.at[pl.ds(off, sz)]))
            ebase = sid * EPT
            issue(0, sv0, dv0, ebase)
            plsc.subcore_barrier()

            def pair(g, state):
                b0 = 2 * g
                wait2()
                issue(b0 + 1, sv1, dv1, ebase)
                state = process(sv0, dv0, state, base)
                wait2()

                @pl.when(b0 + 2 < NSB)
                def _():
                    issue(b0 + 2, sv0, dv0, ebase)

                return process(sv1, dv1, state, base)

            pos, fc = lax.fori_loop(0, NSB // 2, pair, (0, 0))
            pad_tail(pos)
            pos, fc = fire((pos, fc))
            pad_tail(jnp.maximum(pos, 0))
            fire((0, fc))
            pltpu.make_async_copy(rows0, acc_sh.at[didx_f0], sems0).wait()
            pltpu.make_async_copy(rows1, acc_sh.at[didx_f1], sems1).wait()
            plsc.subcore_barrier()
            _per_tile_rows(sid, Q, lambda off, sz: pltpu.sync_copy(
                acc_sh.at[pl.ds(off, sz)], out_hbm.at[pl.ds(base + off, sz)]))
            plsc.subcore_barrier()

    return k(h, src, dst, zeros)


def _scales_tc(counts, ones32):
    def body(c0_ref, c1_ref, o_ref, s0_ref, s1_ref):
        dn = (((0,), (0,)), ((), ()))
        c0 = lax.dot_general(c0_ref[...], o_ref[...], dn,
                             preferred_element_type=_f32)
        c1 = lax.dot_general(c1_ref[...], o_ref[...], dn,
                             preferred_element_type=_f32)
        s0_ref[...] = lax.rsqrt(jnp.maximum(c0, 1.0))
        s1_ref[...] = lax.rsqrt(jnp.maximum(c1, 1.0))

    sds = jax.ShapeDtypeStruct((NN, 1), _f32)
    return pl.pallas_call(
        body,
        grid=(1,),
        in_specs=[
            pl.BlockSpec((NW, NN), lambda j: (0, 0)),
            pl.BlockSpec((NW, NN), lambda j: (0, 0)),
            pl.BlockSpec((NW, 1), lambda j: (0, 0)),
        ],
        out_specs=[pl.BlockSpec((NN, 1), lambda j: (0, 0))] * 2,
        out_shape=[sds, sds],
    )(counts.reshape(2, NW, NN)[0], counts.reshape(2, NW, NN)[1], ones32)


def _h0_tc(feats, s_src):
    def body(x_ref, s_ref, o_ref):
        o_ref[...] = x_ref[...] * s_ref[...]

    return pl.pallas_call(
        body,
        grid=(NN // R,),
        in_specs=[
            pl.BlockSpec((R, DD), lambda j: (j, 0)),
            pl.BlockSpec((R, 1), lambda j: (j, 0)),
        ],
        out_specs=pl.BlockSpec((R, DD), lambda j: (j, 0)),
        out_shape=jax.ShapeDtypeStruct((NN, DD), _f32),
    )(feats, s_src)


def _dense_core(a_ref, sd_ref, w_ref, b_ref, g_ref, be_ref):
    x = a_ref[...] * sd_ref[...]
    y = jnp.dot(x, w_ref[...], preferred_element_type=_f32) + b_ref[...]
    mu = jnp.mean(y, axis=1, keepdims=True)
    d = y - mu
    var = jnp.mean(d * d, axis=1, keepdims=True)
    yn = d * lax.rsqrt(var + 1e-5) * g_ref[...] + be_ref[...]
    return jnp.where(yn > 0, yn, jnp.exp(yn) - 1.0)


_row_blk = pl.BlockSpec((R, 1), lambda j: (j, 0))
_mat_blk = pl.BlockSpec((R, DD), lambda j: (j, 0))
_w_blk = pl.BlockSpec((DD, DD), lambda j: (0, 0))
_v_blk = pl.BlockSpec((1, DD), lambda j: (0, 0))


def _layer_tc(agg, s_dst, s_src, W, b, g, be):
    def body(a_ref, sd_ref, ss_ref, w_ref, b_ref, g_ref, be_ref, o_ref):
        e = _dense_core(a_ref, sd_ref, w_ref, b_ref, g_ref, be_ref)
        o_ref[...] = e * ss_ref[...]

    return pl.pallas_call(
        body,
        grid=(NN // R,),
        in_specs=[_mat_blk, _row_blk, _row_blk, _w_blk, _v_blk, _v_blk, _v_blk],
        out_specs=_mat_blk,
        out_shape=jax.ShapeDtypeStruct((NN, DD), _f32),
    )(agg, s_dst, s_src, W, b, g, be)


def _last_layer_tc(agg, s_dst, W, b, g, be, pwt):
    def body(a_ref, sd_ref, w_ref, b_ref, g_ref, be_ref, pwt_ref, o_ref):
        e = _dense_core(a_ref, sd_ref, w_ref, b_ref, g_ref, be_ref)
        o_ref[...] = jnp.sum(e * pwt_ref[...], axis=1, keepdims=True)

    return pl.pallas_call(
        body,
        grid=(NN // R,),
        in_specs=[_mat_blk, _row_blk, _w_blk, _v_blk, _v_blk, _v_blk,
                  pl.BlockSpec((R, DD), lambda j: (0, 0))],
        out_specs=_row_blk,
        out_shape=jax.ShapeDtypeStruct((NN, 1), _f32),
    )(agg, s_dst, W, b, g, be, pwt)


def _head_tc(rowsums, pb):
    def body(x_ref, pb_ref, o_ref):
        o_ref[...] = jnp.sum(x_ref[...], axis=1, keepdims=True) + pb_ref[...]

    return pl.pallas_call(
        body,
        grid=(NB // RB,),
        in_specs=[pl.BlockSpec((RB, NPER), lambda j: (j, 0)),
                  pl.BlockSpec((1, 1), lambda j: (0, 0))],
        out_specs=pl.BlockSpec((RB, 1), lambda j: (j, 0)),
        out_shape=jax.ShapeDtypeStruct((NB, 1), _f32),
    )(rowsums, pb)


def kernel(features, edge_index, W0, b0, g0, beta0, W1, b1, g1, beta1,
           W2, b2, g2, beta2, pW, pb):
    src, dst = edge_index[0], edge_index[1]
    feats = features.reshape(NN, DD)
    zeros = jnp.zeros((ZR, DD), _f32)
    ones32 = jnp.ones((NW, 1), _f32)
    pwt = jnp.tile(pW.reshape(NPER, DD), (R // NPER, 1))

    counts = _degrees_sc(src, dst)
    s_src, s_dst = _scales_tc(counts, ones32)

    h = _h0_tc(feats, s_src)
    for W, b, g, be in ((W0, b0, g0, beta0), (W1, b1, g1, beta1)):
        agg = _segsum_sc(h, src, dst, zeros)
        h = _layer_tc(agg, s_dst, s_src, W, b.reshape(1, DD),
                      g.reshape(1, DD), be.reshape(1, DD))
    agg = _segsum_sc(h, src, dst, zeros)
    rowsums = _last_layer_tc(agg, s_dst, W2, b2.reshape(1, DD),
                             g2.reshape(1, DD), beta2.reshape(1, DD), pwt)
    return _head_tc(rowsums.reshape(NB, NPER), pb.reshape(1, 1))


# speedup vs baseline: 3.0093x; 1.0509x over previous
import functools

import jax
import jax.numpy as jnp
from jax import lax
from jax.experimental import pallas as pl
from jax.experimental.pallas import tpu as pltpu
from jax.experimental.pallas import tpu_sc as plsc

NB, NPER, DD = 4000, 9, 128
NN = NB * NPER
EE = 576000
NC, NS = 2, 16
NW = NC * NS
Q = NN // 4
QP = Q + 8
K = 96
FG = 5
CAP = K + 16 * FG
IB = 1200
EPT = EE // NS
EPW = EE // NW
DB = 720
ZR = 568
R = 288
RC = 1200
RB = 200

_f32 = jnp.float32


def _mesh():
    return plsc.VectorSubcoreMesh(
        core_axis_name="c", subcore_axis_name="s",
        num_cores=NC, num_subcores=NS)


def _per_tile_rows(sid, total, fn):
    @pl.when(sid < NS - 1)
    def _():
        fn(sid * ZR, ZR)

    @pl.when(sid == NS - 1)
    def _():
        fn((NS - 1) * ZR, total - (NS - 1) * ZR)


def _degrees_sc(src, dst):
    @functools.partial(
        pl.kernel,
        out_type=jax.ShapeDtypeStruct((2 * NW, NN), _f32),
        mesh=_mesh(),
        compiler_params=pltpu.CompilerParams(needs_layout_passes=False),
        scratch_types=[
            pltpu.VMEM((DB,), jnp.int32),
            pltpu.VMEM((DB,), jnp.int32),
            pltpu.VMEM((NN,), _f32),
            pltpu.VMEM((NN,), _f32),
        ],
    )
    def k(src_hbm, dst_hbm, out_hbm, sv, dv, cs, cd):
        cid = lax.axis_index("c")
        sid = lax.axis_index("s")
        wid = sid * NC + cid
        zero16 = jnp.zeros((16,), _f32)
        one16 = jnp.ones((16,), _f32)

        def zb(i, carry):
            cs[pl.ds(i * 16, 16)] = zero16
            cd[pl.ds(i * 16, 16)] = zero16
            return carry

        lax.fori_loop(0, NN // 16, zb, 0)
        ebase = wid * EPW

        def body(b, carry):
            base = ebase + b * DB
            pltpu.sync_copy(src_hbm.at[pl.ds(base, DB)], sv)
            pltpu.sync_copy(dst_hbm.at[pl.ds(base, DB)], dv)
            for j in range(DB // 16):
                plsc.addupdate_scatter(cs, [sv[pl.ds(j * 16, 16)]], one16)
                plsc.addupdate_scatter(cd, [dv[pl.ds(j * 16, 16)]], one16)
            return carry

        lax.fori_loop(0, EPW // DB, body, 0)
        pltpu.sync_copy(cs, out_hbm.at[wid])
        pltpu.sync_copy(cd, out_hbm.at[NW + wid])

    return k(src, dst)


def _segsum_sc(h, src, dst, zeros):
    @functools.partial(
        pl.kernel,
        out_type=jax.ShapeDtypeStruct((NN, DD), _f32),
        mesh=_mesh(),
        compiler_params=pltpu.CompilerParams(needs_layout_passes=False),
        scratch_types=[
            pltpu.VMEM((IB,), jnp.int32),
            pltpu.VMEM((IB,), jnp.int32),
            pltpu.VMEM((IB,), jnp.int32),
            pltpu.VMEM((IB,), jnp.int32),
            pltpu.VMEM((CAP,), jnp.int32),
            pltpu.VMEM((CAP,), jnp.int32),
            pltpu.VMEM((K,), jnp.int32),
            pltpu.VMEM((K,), jnp.int32),
            pltpu.VMEM((K,), jnp.int32),
            pltpu.VMEM((K,), jnp.int32),
            pltpu.VMEM((K, DD), _f32),
            pltpu.VMEM((K, DD), _f32),
            pltpu.VMEM_SHARED((QP, DD), _f32),
            pltpu.SemaphoreType.DMA,
            pltpu.SemaphoreType.DMA,
            pltpu.SemaphoreType.DMA,
            pltpu.SemaphoreType.DMA,
        ],
    )
    def k(h_hbm, src_hbm, dst_hbm, zeros_hbm, out_hbm,
          sv0, dv0, sv1, dv1, ssta, dsta, sidx_f0, didx_f0, sidx_f1, didx_f1,
          rows0, rows1, acc_sh, semg, semi, sems0, sems1):
        cid = lax.axis_index("c")
        sid = lax.axis_index("s")
        G = K // 16

        def fire(args):
            p, fc = args

            def slot(sidx_f, didx_f, rows, sems):
                @pl.when(fc >= 2)
                def _():
                    pltpu.make_async_copy(
                        rows, acc_sh.at[didx_f], sems).wait()

                for j in range(G):
                    sidx_f[pl.ds(j * 16, 16)] = ssta[pl.ds(j * 16, 16)]
                    didx_f[pl.ds(j * 16, 16)] = dsta[pl.ds(j * 16, 16)]
                pltpu.async_copy(h_hbm.at[sidx_f], rows, semg).wait()
                pltpu.async_copy(rows, acc_sh.at[didx_f], sems, add=True)

            @pl.when((fc & 1) == 0)
            def _():
                slot(sidx_f0, didx_f0, rows0, sems0)

            @pl.when((fc & 1) == 1)
            def _():
                slot(sidx_f1, didx_f1, rows1, sems1)

            for j in range((CAP - K) // 16):
                ssta[pl.ds(j * 16, 16)] = ssta[pl.ds(K + j * 16, 16)]
                dsta[pl.ds(j * 16, 16)] = dsta[pl.ds(K + j * 16, 16)]
            return p - K, fc + 1

        def pad_tail(limit):
            for j in range(CAP // 16):
                lane = lax.iota(jnp.int32, 16) + j * 16
                pad = lane >= limit
                ssta[pl.ds(j * 16, 16)] = jnp.where(
                    pad, 0, ssta[pl.ds(j * 16, 16)])
                dsta[pl.ds(j * 16, 16)] = jnp.where(
                    pad, Q, dsta[pl.ds(j * 16, 16)])

        def issue(b, s_buf, d_buf, ebase):
            bb = ebase + b * IB
            pltpu.async_copy(src_hbm.at[pl.ds(bb, IB)], s_buf, semi)
            pltpu.async_copy(dst_hbm.at[pl.ds(bb, IB)], d_buf, semi)

        def wait2():
            pltpu.make_async_copy(src_hbm.at[pl.ds(0, IB)], sv0, semi).wait()
            pltpu.make_async_copy(src_hbm.at[pl.ds(0, IB)], dv0, semi).wait()

        def process(s_buf, d_buf, state, base):
            for jj in range(IB // (16 * FG)):
                pos, fc = state
                for j in range(FG):
                    o = (jj * FG + j) * 16
                    s16 = s_buf[pl.ds(o, 16)]
                    d16 = d_buf[pl.ds(o, 16)]
                    ok = (d16 >= base) & (d16 < base + Q)
                    plsc.store_compressed(ssta.at[pl.ds(pos, 16)], s16,
                                          mask=ok)
                    plsc.store_compressed(dsta.at[pl.ds(pos, 16)],
                                          d16 - base, mask=ok)
                    pos = pos + jnp.sum(ok.astype(jnp.int32))
                state = lax.cond(pos >= K, fire, lambda a: a, (pos, fc))
            return state

        NSB = EPT // IB
        for r in range(2):
            base = pl.multiple_of((2 * r + cid) * Q, 8)
            _per_tile_rows(sid, QP, lambda off, sz: pltpu.sync_copy(
                zeros_hbm.at[pl.ds(0, sz)], acc_sh.at[pl.ds(off, sz)]))
            ebase = sid * EPT
            issue(0, sv0, dv0, ebase)
            plsc.subcore_barrier()

            def pair(g, state):
                b0 = 2 * g
                wait2()
                issue(b0 + 1, sv1, dv1, ebase)
                state = process(sv0, dv0, state, base)
                wait2()

                @pl.when(b0 + 2 < NSB)
                def _():
                    issue(b0 + 2, sv0, dv0, ebase)

                return process(sv1, dv1, state, base)

            pos, fc = lax.fori_loop(0, NSB // 2, pair, (0, 0))
            pad_tail(pos)
            pos, fc = fire((pos, fc))
            pad_tail(jnp.maximum(pos, 0))
            fire((0, fc))
            pltpu.make_async_copy(rows0, acc_sh.at[didx_f0], sems0).wait()
            pltpu.make_async_copy(rows1, acc_sh.at[didx_f1], sems1).wait()
            plsc.subcore_barrier()
            _per_tile_rows(sid, Q, lambda off, sz: pltpu.sync_copy(
                acc_sh.at[pl.ds(off, sz)], out_hbm.at[pl.ds(base + off, sz)]))
            plsc.subcore_barrier()

    return k(h, src, dst, zeros)


def _scales_tc(counts, ones32):
    def body(c0_ref, c1_ref, o_ref, s0_ref, s1_ref):
        dn = (((0,), (0,)), ((), ()))
        c0 = lax.dot_general(c0_ref[...], o_ref[...], dn,
                             preferred_element_type=_f32)
        c1 = lax.dot_general(c1_ref[...], o_ref[...], dn,
                             preferred_element_type=_f32)
        s0_ref[...] = lax.rsqrt(jnp.maximum(c0, 1.0))
        s1_ref[...] = lax.rsqrt(jnp.maximum(c1, 1.0))

    sds = jax.ShapeDtypeStruct((NN, 1), _f32)
    return pl.pallas_call(
        body,
        grid=(1,),
        in_specs=[
            pl.BlockSpec((NW, NN), lambda j: (0, 0)),
            pl.BlockSpec((NW, NN), lambda j: (0, 0)),
            pl.BlockSpec((NW, 1), lambda j: (0, 0)),
        ],
        out_specs=[pl.BlockSpec((NN, 1), lambda j: (0, 0))] * 2,
        out_shape=[sds, sds],
    )(counts.reshape(2, NW, NN)[0], counts.reshape(2, NW, NN)[1], ones32)


def _h0_tc(feats, s_src):
    def body(x_ref, s_ref, o_ref):
        o_ref[...] = x_ref[...] * s_ref[...]

    return pl.pallas_call(
        body,
        grid=(NN // R,),
        in_specs=[
            pl.BlockSpec((R, DD), lambda j: (j, 0)),
            pl.BlockSpec((R, 1), lambda j: (j, 0)),
        ],
        out_specs=pl.BlockSpec((R, DD), lambda j: (j, 0)),
        out_shape=jax.ShapeDtypeStruct((NN, DD), _f32),
    )(feats, s_src)


def _dense_core(a_ref, sd_ref, w_ref, b_ref, g_ref, be_ref):
    x = a_ref[...] * sd_ref[...]
    y = jnp.dot(x.astype(jnp.bfloat16), w_ref[...].astype(jnp.bfloat16),
                preferred_element_type=_f32) + b_ref[...]
    mu = jnp.mean(y, axis=1, keepdims=True)
    d = y - mu
    var = jnp.mean(d * d, axis=1, keepdims=True)
    yn = d * lax.rsqrt(var + 1e-5) * g_ref[...] + be_ref[...]
    return jnp.where(yn > 0, yn, jnp.exp(yn) - 1.0)


_row_blk = pl.BlockSpec((R, 1), lambda j: (j, 0))
_mat_blk = pl.BlockSpec((R, DD), lambda j: (j, 0))
_w_blk = pl.BlockSpec((DD, DD), lambda j: (0, 0))
_v_blk = pl.BlockSpec((1, DD), lambda j: (0, 0))


def _layer_tc(agg, s_dst, s_src, W, b, g, be):
    def body(a_ref, sd_ref, ss_ref, w_ref, b_ref, g_ref, be_ref, o_ref):
        e = _dense_core(a_ref, sd_ref, w_ref, b_ref, g_ref, be_ref)
        o_ref[...] = e * ss_ref[...]

    return pl.pallas_call(
        body,
        grid=(NN // R,),
        in_specs=[_mat_blk, _row_blk, _row_blk, _w_blk, _v_blk, _v_blk, _v_blk],
        out_specs=_mat_blk,
        out_shape=jax.ShapeDtypeStruct((NN, DD), _f32),
    )(agg, s_dst, s_src, W, b, g, be)


def _last_layer_tc(agg, s_dst, W, b, g, be, pwt):
    def body(a_ref, sd_ref, w_ref, b_ref, g_ref, be_ref, pwt_ref, o_ref):
        e = _dense_core(a_ref, sd_ref, w_ref, b_ref, g_ref, be_ref)
        e16 = e.astype(jnp.bfloat16).astype(_f32)
        p16 = pwt_ref[...].astype(jnp.bfloat16).astype(_f32)
        o_ref[...] = jnp.sum(e16 * p16, axis=1, keepdims=True)

    return pl.pallas_call(
        body,
        grid=(NN // R,),
        in_specs=[_mat_blk, _row_blk, _w_blk, _v_blk, _v_blk, _v_blk,
                  pl.BlockSpec((R, DD), lambda j: (0, 0))],
        out_specs=_row_blk,
        out_shape=jax.ShapeDtypeStruct((NN, 1), _f32),
    )(agg, s_dst, W, b, g, be, pwt)


def _head_tc(rowsums, pb):
    def body(x_ref, pb_ref, o_ref):
        o_ref[...] = jnp.sum(x_ref[...], axis=1, keepdims=True) + pb_ref[...]

    return pl.pallas_call(
        body,
        grid=(NB // RB,),
        in_specs=[pl.BlockSpec((RB, NPER), lambda j: (j, 0)),
                  pl.BlockSpec((1, 1), lambda j: (0, 0))],
        out_specs=pl.BlockSpec((RB, 1), lambda j: (j, 0)),
        out_shape=jax.ShapeDtypeStruct((NB, 1), _f32),
    )(rowsums, pb)


def kernel(features, edge_index, W0, b0, g0, beta0, W1, b1, g1, beta1,
           W2, b2, g2, beta2, pW, pb):
    src, dst = edge_index[0], edge_index[1]
    feats = features.reshape(NN, DD)
    zeros = jnp.zeros((ZR, DD), _f32)
    ones32 = jnp.ones((NW, 1), _f32)
    pwt = jnp.tile(pW.reshape(NPER, DD), (R // NPER, 1))

    counts = _degrees_sc(src, dst)
    s_src, s_dst = _scales_tc(counts, ones32)

    h = _h0_tc(feats, s_src)
    for W, b, g, be in ((W0, b0, g0, beta0), (W1, b1, g1, beta1)):
        agg = _segsum_sc(h, src, dst, zeros)
        h = _layer_tc(agg, s_dst, s_src, W, b.reshape(1, DD),
                      g.reshape(1, DD), be.reshape(1, DD))
    agg = _segsum_sc(h, src, dst, zeros)
    rowsums = _last_layer_tc(agg, s_dst, W2, b2.reshape(1, DD),
                             g2.reshape(1, DD), beta2.reshape(1, DD), pwt)
    return _head_tc(rowsums.reshape(NB, NPER), pb.reshape(1, 1))
